# Initial kernel scaffold; baseline (speedup 1.0000x reference)
import functools

import jax
import jax.numpy as jnp
from jax import lax
from jax.experimental import pallas as pl
from jax.experimental.pallas import tpu as pltpu
from jax.experimental.pallas import tpu_sc as plsc

N = 1000000
D0 = 512
PLANE = D0 * D0
CHUNK = 512
SUB = 128
NSUB = CHUNK // SUB
NCHUNKS = (N + CHUNK - 1) // CHUNK
NW = 32
KMAX = (NCHUNKS + NW - 1) // NW


def _body(xyz_hbm, table_hbm, out_hbm,
          xyz_v, idx_v, fr_v, rows_v, out_v, sem):
    cid = lax.axis_index("c")
    sid = lax.axis_index("s")
    wid = sid * 2 + cid

    iota = lax.iota(jnp.int32, 16)

    def chunk_body(k, _):
        cc = wid + k * NW

        @pl.when(cc < NCHUNKS)
        def _():
            base = jnp.minimum(cc * CHUNK, N - CHUNK)
            pltpu.sync_copy(xyz_hbm.at[pl.ds(base, CHUNK)], xyz_v)

            def p1(g, _):
                i16 = g * 16 + iota
                xv = plsc.load_gather(xyz_v, [i16, jnp.full((16,), 0, jnp.int32)])
                yv = plsc.load_gather(xyz_v, [i16, jnp.full((16,), 1, jnp.int32)])
                zv = plsc.load_gather(xyz_v, [i16, jnp.full((16,), 2, jnp.int32)])
                X = xv * 255.5 + 255.5
                Y = yv * 255.5 + 255.5
                Z = zv * 5110.0 + 255.5

                def coords(s):
                    ti = s.astype(jnp.int32)
                    tf = ti.astype(jnp.float32)
                    fl = jnp.where(s < tf, tf - 1.0, tf)
                    fr = s - fl
                    ci = jnp.minimum(jnp.maximum(ti, 0), D0 - 2)
                    return fr, ci

                frX, ciX = coords(X)
                frY, ciY = coords(Y)
                frZ, ciZ = coords(Z)

                sl = pl.ds(g * 16, 16)
                idx_v[0, sl] = ciX * D0 + ciY
                idx_v[1, sl] = ciY * D0 + ciZ + 2 * PLANE
                idx_v[2, sl] = ciZ * D0 + ciX + 1 * PLANE
                fr_v[0, sl] = frX
                fr_v[1, sl] = frY
                fr_v[2, sl] = frZ
                return 0

            lax.fori_loop(0, CHUNK // 16, p1, 0)

            copies = []
            for p in range(3):
                for j in range(NSUB):
                    copies.append(pltpu.async_copy(
                        table_hbm.at[idx_v.at[p, pl.ds(j * SUB, SUB)]],
                        rows_v.at[p, pl.ds(j * SUB, SUB)],
                        sem))
            for cpy in copies:
                cpy.wait()

            def p3(g, _):
                i16 = g * 16 + iota
                sl = pl.ds(g * 16, 16)
                ru3 = (fr_v[0, sl], fr_v[1, sl], fr_v[2, sl])
                for p in range(3):
                    ru = ru3[p]
                    rv = ru3[(p + 1) % 3]
                    gu = 1.0 - ru
                    gv = 1.0 - rv
                    w00 = gu * gv
                    w01 = gu * rv
                    w10 = ru * gv
                    w11 = ru * rv
                    pv = jnp.full((16,), p, jnp.int32)
                    for c in range(4):
                        v00 = plsc.load_gather(rows_v, [pv, i16, jnp.full((16,), c, jnp.int32)])
                        v01 = plsc.load_gather(rows_v, [pv, i16, jnp.full((16,), 4 + c, jnp.int32)])
                        v10 = plsc.load_gather(rows_v, [pv, i16, jnp.full((16,), 8 + c, jnp.int32)])
                        v11 = plsc.load_gather(rows_v, [pv, i16, jnp.full((16,), 12 + c, jnp.int32)])
                        acc = w00 * v00 + w01 * v01 + w10 * v10 + w11 * v11
                        plsc.store_scatter(out_v, [i16, jnp.full((16,), 4 * p + c, jnp.int32)], acc)
                return 0

            lax.fori_loop(0, CHUNK // 16, p3, 0)

            pltpu.sync_copy(out_v, out_hbm.at[pl.ds(base, CHUNK)])

        return 0

    lax.fori_loop(0, KMAX, chunk_body, 0)


def kernel(xyz, feature_maps):
    fmT = jnp.transpose(feature_maps, (0, 2, 3, 1))
    packed = jnp.concatenate(
        [fmT,
         jnp.roll(fmT, -1, axis=2),
         jnp.roll(fmT, -1, axis=1),
         jnp.roll(jnp.roll(fmT, -1, axis=1), -1, axis=2)],
        axis=-1).reshape(3 * PLANE, 16)

    mesh = plsc.VectorSubcoreMesh(core_axis_name="c", subcore_axis_name="s")
    run = functools.partial(
        pl.kernel, _body, mesh=mesh,
        out_type=jax.ShapeDtypeStruct((N, 12), jnp.float32),
        scratch_types=[
            pltpu.VMEM((CHUNK, 3), jnp.float32),
            pltpu.VMEM((3, CHUNK), jnp.int32),
            pltpu.VMEM((3, CHUNK), jnp.float32),
            pltpu.VMEM((3, CHUNK, 16), jnp.float32),
            pltpu.VMEM((CHUNK, 12), jnp.float32),
            pltpu.SemaphoreType.DMA,
        ])()
    return run(xyz, packed)


# speedup vs baseline: 24.4592x; 24.4592x over previous
import jax
import jax.numpy as jnp
from jax import lax
from jax.experimental import pallas as pl
from jax.experimental.pallas import tpu as pltpu
from jax.experimental.pallas import tpu_sc as plsc

N = 1000000
D0 = 512
NOCT = D0 // 8
PLANE_ROWS = D0 * NOCT
CHUNK = 128
SUB = 128
NSUB = CHUNK // SUB
NCHUNKS = (N + CHUNK - 1) // CHUNK
NW = 32
KMAX = (NCHUNKS + NW - 1) // NW


def _body(xyz_hbm, table_hbm, out_hbm,
          xyz_v, idxA_v, idxB_v, idxC_v, frX_v, frY_v, frZ_v,
          octA_v, octB_v, octC_v, rowsA_v, rowsB_v, rowsC_v, out_v, sem):
    cid = lax.axis_index("c")
    sid = lax.axis_index("s")
    wid = sid * 2 + cid

    iota = lax.iota(jnp.int32, 16)

    def chunk_body(k, _):
        cc = wid + k * NW

        @pl.when(cc < NCHUNKS)
        def _():
            base = jnp.minimum(cc * CHUNK, N - CHUNK)
            pltpu.sync_copy(xyz_hbm.at[pl.ds(base, CHUNK)], xyz_v)

            def p1(g, _):
                i16 = g * 16 + iota
                xv = plsc.load_gather(xyz_v, [i16, jnp.full((16,), 0, jnp.int32)])
                yv = plsc.load_gather(xyz_v, [i16, jnp.full((16,), 1, jnp.int32)])
                zv = plsc.load_gather(xyz_v, [i16, jnp.full((16,), 2, jnp.int32)])
                X = ((xv + 1.0) * 511.0) * 0.5
                Y = ((yv + 1.0) * 511.0) * 0.5
                Z = ((zv / 0.05 + 1.0) * 511.0) * 0.5

                def coords(s):
                    ti = s.astype(jnp.int32)
                    tf = ti.astype(jnp.float32)
                    fl = jnp.where(s < tf, tf - 1.0, tf)
                    fr = s - fl
                    ci = jnp.minimum(jnp.maximum(ti, 0), D0 - 2)
                    return fr, ci

                frX, ciX = coords(X)
                frY, ciY = coords(Y)
                frZ, ciZ = coords(Z)

                sl = pl.ds(g * 16, 16)
                idxA_v[sl] = ciX * NOCT + (ciY >> 3)
                idxB_v[sl] = ciY * NOCT + (ciZ >> 3) + 2 * PLANE_ROWS
                idxC_v[sl] = ciZ * NOCT + (ciX >> 3) + 1 * PLANE_ROWS
                octA_v[sl] = (ciY & 7) * 16
                octB_v[sl] = (ciZ & 7) * 16
                octC_v[sl] = (ciX & 7) * 16
                frX_v[sl] = frX
                frY_v[sl] = frY
                frZ_v[sl] = frZ
                return 0

            lax.fori_loop(0, CHUNK // 16, p1, 0)

            copies = []
            for idx_r, rows_r in ((idxA_v, rowsA_v), (idxB_v, rowsB_v),
                                  (idxC_v, rowsC_v)):
                copies.append(pltpu.async_copy(
                    table_hbm.at[idx_r], rows_r, sem))
            for cpy in copies:
                cpy.wait()

            def p3(g, _):
                i16 = g * 16 + iota
                sl = pl.ds(g * 16, 16)
                ru3 = (frX_v[sl], frY_v[sl], frZ_v[sl])
                oct3 = (octA_v[sl], octB_v[sl], octC_v[sl])
                rows3 = (rowsA_v, rowsB_v, rowsC_v)
                for p in range(3):
                    ru = ru3[p]
                    rv = ru3[(p + 1) % 3]
                    rows_r = rows3[p]
                    off = oct3[p]
                    gu = 1.0 - ru
                    gv = 1.0 - rv
                    w00 = gu * gv
                    w01 = gu * rv
                    w10 = ru * gv
                    w11 = ru * rv
                    for c in range(4):
                        v00 = plsc.load_gather(rows_r, [i16, off + c])
                        v01 = plsc.load_gather(rows_r, [i16, off + (4 + c)])
                        v10 = plsc.load_gather(rows_r, [i16, off + (8 + c)])
                        v11 = plsc.load_gather(rows_r, [i16, off + (12 + c)])
                        acc = ((w00 * v00 + w10 * v10) + w01 * v01) + w11 * v11
                        plsc.store_scatter(out_v, [i16, jnp.full((16,), 4 * p + c, jnp.int32)], acc)
                return 0

            lax.fori_loop(0, CHUNK // 16, p3, 0)

            pltpu.sync_copy(out_v, out_hbm.at[pl.ds(base, CHUNK)])

        return 0

    lax.fori_loop(0, KMAX, chunk_body, 0)


def kernel(xyz, feature_maps):
    fmT = jnp.transpose(feature_maps, (0, 2, 3, 1))
    packed = jnp.concatenate(
        [fmT,
         jnp.roll(fmT, -1, axis=2),
         jnp.roll(fmT, -1, axis=1),
         jnp.roll(jnp.roll(fmT, -1, axis=1), -1, axis=2)],
        axis=-1)
    table = packed.reshape(3 * PLANE_ROWS, 128)

    mesh = plsc.VectorSubcoreMesh(core_axis_name="c", subcore_axis_name="s")
    run = pl.kernel(
        _body, mesh=mesh,
        out_type=jax.ShapeDtypeStruct((N, 12), jnp.float32),
        compiler_params=pltpu.CompilerParams(needs_layout_passes=False),
        scratch_types=[
            pltpu.VMEM((CHUNK, 3), jnp.float32),
            pltpu.VMEM((CHUNK,), jnp.int32),
            pltpu.VMEM((CHUNK,), jnp.int32),
            pltpu.VMEM((CHUNK,), jnp.int32),
            pltpu.VMEM((CHUNK,), jnp.float32),
            pltpu.VMEM((CHUNK,), jnp.float32),
            pltpu.VMEM((CHUNK,), jnp.float32),
            pltpu.VMEM((CHUNK,), jnp.int32),
            pltpu.VMEM((CHUNK,), jnp.int32),
            pltpu.VMEM((CHUNK,), jnp.int32),
            pltpu.VMEM((CHUNK, 128), jnp.float32),
            pltpu.VMEM((CHUNK, 128), jnp.float32),
            pltpu.VMEM((CHUNK, 128), jnp.float32),
            pltpu.VMEM((CHUNK, 12), jnp.float32),
            pltpu.SemaphoreType.DMA,
        ])
    return run(xyz, table)
